# Initial kernel scaffold; baseline (speedup 1.0000x reference)
import functools

import jax
import jax.numpy as jnp
from jax import lax
from jax.experimental import pallas as pl
from jax.experimental.pallas import tpu as pltpu
from jax.experimental.pallas import tpu_sc as plsc

N = 10000
E = 320000
D = 128

NC = 2
NS = 16
NW = NC * NS
EPW = E // NW
K = 80
CHUNKS_PER_W = EPW // K
ROWS_PER_TILE = N // NS
ZB = 125


def _linear_body(x_ref, wt_ref, b_ref, o_ref):
    o_ref[...] = (
        jnp.dot(x_ref[...], wt_ref[...], preferred_element_type=jnp.float32)
        + b_ref[...]
    )


_BM = 1000

_linear = pl.pallas_call(
    _linear_body,
    grid=(N // _BM,),
    in_specs=[
        pl.BlockSpec((_BM, D), lambda i: (i, 0)),
        pl.BlockSpec((D, D), lambda i: (0, 0)),
        pl.BlockSpec((1, D), lambda i: (0, 0)),
    ],
    out_specs=pl.BlockSpec((_BM, D), lambda i: (i, 0)),
    out_shape=jax.ShapeDtypeStruct((N, D), jnp.float32),
)


def _spmm_body(out_hbm, row_hbm, col_hbm, val_hbm, part_hbm,
               col_all, row_all, val_all, rows_v, zbuf, acc, sem):
    cid = lax.axis_index("c")
    sid = lax.axis_index("s")
    wid = sid * NC + cid

    zero16 = jnp.zeros((16,), jnp.float32)

    def zfill(i, carry):
        for j in range(D // 16):
            zbuf[i, pl.ds(j * 16, 16)] = zero16
        return carry

    lax.fori_loop(0, ZB, zfill, 0)
    for t in range(ROWS_PER_TILE // ZB):
        pltpu.sync_copy(zbuf, acc.at[pl.ds(sid * ROWS_PER_TILE + t * ZB, ZB)])

    c0 = wid * CHUNKS_PER_W
    pltpu.sync_copy(col_hbm.at[pl.ds(c0, CHUNKS_PER_W)], col_all)
    pltpu.sync_copy(row_hbm.at[pl.ds(c0, CHUNKS_PER_W)], row_all)
    pltpu.sync_copy(val_hbm.at[pl.ds(c0, CHUNKS_PER_W)], val_all)

    plsc.subcore_barrier()

    def chunk_body(i, carry):
        pltpu.async_copy(out_hbm.at[col_all.at[i]], rows_v, sem).wait()

        def edge_body(e, c2):
            v = plsc.load_gather(
                val_all,
                [jnp.full((16,), i, jnp.int32), jnp.full((16,), e, jnp.int32)],
            )
            for j in range(D // 16):
                sl = pl.ds(j * 16, 16)
                rows_v[e, sl] = rows_v[e, sl] * v
            return c2

        lax.fori_loop(0, K, edge_body, 0)
        pltpu.sync_copy(rows_v, acc.at[row_all.at[i]], add=True)
        return carry

    lax.fori_loop(0, CHUNKS_PER_W, chunk_body, 0)

    plsc.subcore_barrier()

    r0 = sid * ROWS_PER_TILE
    pltpu.sync_copy(acc.at[pl.ds(r0, ROWS_PER_TILE)],
                    part_hbm.at[cid, pl.ds(r0, ROWS_PER_TILE)])


_spmm = functools.partial(
    pl.kernel,
    out_type=jax.ShapeDtypeStruct((NC, N, D), jnp.float32),
    mesh=plsc.VectorSubcoreMesh(core_axis_name="c", subcore_axis_name="s"),
    scratch_types=[
        pltpu.VMEM((CHUNKS_PER_W, K), jnp.int32),
        pltpu.VMEM((CHUNKS_PER_W, K), jnp.int32),
        pltpu.VMEM((CHUNKS_PER_W, K), jnp.float32),
        pltpu.VMEM((K, D), jnp.float32),
        pltpu.VMEM((ZB, D), jnp.float32),
        pltpu.VMEM_SHARED((N, D), jnp.float32),
        pltpu.SemaphoreType.DMA,
    ],
)(_spmm_body)


def _add_body(p_ref, o_ref):
    o_ref[...] = p_ref[0] + p_ref[1]


_combine = pl.pallas_call(
    _add_body,
    grid=(N // _BM,),
    in_specs=[pl.BlockSpec((NC, _BM, D), lambda i: (0, i, 0))],
    out_specs=pl.BlockSpec((_BM, D), lambda i: (i, 0)),
    out_shape=jax.ShapeDtypeStruct((N, D), jnp.float32),
)


def kernel(x, adj_indices, adj_values, W, b):
    out = _linear(x, W.T, b.reshape(1, D))
    row = adj_indices[0].reshape(E // K, K)
    col = adj_indices[1].reshape(E // K, K)
    val = adj_values.reshape(E // K, K)
    parts = _spmm(out, row, col, val)
    return _combine(parts)


# speedup vs baseline: 5.4659x; 5.4659x over previous
import functools

import jax
import jax.numpy as jnp
from jax import lax
from jax.experimental import pallas as pl
from jax.experimental.pallas import tpu as pltpu
from jax.experimental.pallas import tpu_sc as plsc

N = 10000
E = 320000
D = 128

NC = 2
NS = 16
HALF = 5120
ACC_H = HALF + 8
RPT = HALF // NS
EPT = E // NS
STAGE = 2000
NSTAGE = EPT // STAGE
GPS = STAGE // 16
CAP = 20224
K = 80


def _linear_body(x_ref, wt_ref, b_ref, o_ref):
    o_ref[...] = (
        jnp.dot(x_ref[...], wt_ref[...], preferred_element_type=jnp.float32)
        + b_ref[...]
    )


_BM = 1000

_linear = pl.pallas_call(
    _linear_body,
    grid=(N // _BM,),
    in_specs=[
        pl.BlockSpec((_BM, D), lambda i: (i, 0)),
        pl.BlockSpec((D, D), lambda i: (0, 0)),
        pl.BlockSpec((1, D), lambda i: (0, 0)),
    ],
    out_specs=pl.BlockSpec((_BM, D), lambda i: (i, 0)),
    out_shape=jax.ShapeDtypeStruct((N, D), jnp.float32),
)


def _spmm_body(out_hbm, row_hbm, col_hbm, val_hbm, res_hbm,
               crow, ccol, cval, rrow, rcol, rval, rows_v, acc, sem, sem2):
    cid = lax.axis_index("c")
    sid = lax.axis_index("s")
    lo = cid * HALF

    zero16 = jnp.zeros((16,), jnp.float32)

    def zfill(r, carry):
        for j in range(D // 16):
            rows_v[r, pl.ds(j * 16, 16)] = zero16
        return carry

    lax.fori_loop(0, K, zfill, 0)
    for t in range(RPT // K):
        pltpu.sync_copy(rows_v, acc.at[pl.ds(sid * RPT + t * K, K)])

    @pl.when(sid == NS - 1)
    def _():
        pltpu.sync_copy(rows_v.at[pl.ds(0, 8)], acc.at[pl.ds(HALF, 8)])

    dummy_row = jnp.full((16,), HALF, jnp.int32)
    zero_i = jnp.zeros((16,), jnp.int32)

    def pfill(p, carry):
        sl = pl.ds(p * 16, 16)
        crow[sl] = dummy_row
        ccol[sl] = zero_i
        cval[sl] = zero16
        return carry

    lax.fori_loop(0, CAP // 16, pfill, 0)

    def stage_body(t, cnt):
        base = sid * EPT + t * STAGE
        pltpu.sync_copy(row_hbm.at[pl.ds(base, STAGE)], rrow)
        pltpu.sync_copy(col_hbm.at[pl.ds(base, STAGE)], rcol)
        pltpu.sync_copy(val_hbm.at[pl.ds(base, STAGE)], rval)

        def group_body(g, cnt):
            sl = pl.ds(g * 16, 16)
            rr = rrow[sl] - lo
            mask = (rr >= 0) & (rr < HALF)
            cs = plsc.cumsum(mask.astype(jnp.int32))
            pos = cnt + cs - 1
            plsc.store_scatter(crow, [pos], rr, mask=mask)
            plsc.store_scatter(ccol, [pos], rcol[sl], mask=mask)
            plsc.store_scatter(cval, [pos], rval[sl], mask=mask)
            return cnt + cs[15]

        return lax.fori_loop(0, GPS, group_body, cnt)

    cnt = lax.fori_loop(0, NSTAGE, stage_body, jnp.int32(0))

    plsc.subcore_barrier()

    nch = (cnt + (K - 1)) // K

    def chunk_body(i, carry):
        b = i * K
        pltpu.async_copy(out_hbm.at[ccol.at[pl.ds(b, K)]], rows_v, sem).wait()

        for g in range(K // 16):
            val16 = cval[pl.ds(b + g * 16, 16)]
            for l in range(16):
                v = jnp.full((16,), val16[l], jnp.float32)
                e = g * 16 + l
                for j in range(D // 16):
                    sl = pl.ds(j * 16, 16)
                    rows_v[e, sl] = rows_v[e, sl] * v

        handles = []
        for g in range(K // 16):
            ridx = crow[pl.ds(b + g * 16, 16)]
            handles.append(
                pltpu.async_copy(rows_v.at[pl.ds(g * 16, 16)],
                                 acc.at[ridx], sem2, add=True)
            )
        for h in handles:
            h.wait()
        return carry

    lax.fori_loop(0, nch, chunk_body, 0)

    plsc.subcore_barrier()

    pltpu.sync_copy(acc.at[pl.ds(sid * RPT, RPT)],
                    res_hbm.at[pl.ds(lo + sid * RPT, RPT)])


_spmm = functools.partial(
    pl.kernel,
    out_type=jax.ShapeDtypeStruct((NC * HALF, D), jnp.float32),
    mesh=plsc.VectorSubcoreMesh(core_axis_name="c", subcore_axis_name="s"),
    compiler_params=pltpu.CompilerParams(needs_layout_passes=False),
    scratch_types=[
        pltpu.VMEM((CAP,), jnp.int32),
        pltpu.VMEM((CAP,), jnp.int32),
        pltpu.VMEM((CAP,), jnp.float32),
        pltpu.VMEM((STAGE,), jnp.int32),
        pltpu.VMEM((STAGE,), jnp.int32),
        pltpu.VMEM((STAGE,), jnp.float32),
        pltpu.VMEM((K, D), jnp.float32),
        pltpu.VMEM_SHARED((ACC_H, D), jnp.float32),
        pltpu.SemaphoreType.DMA,
        pltpu.SemaphoreType.DMA,
    ],
)(_spmm_body)


def kernel(x, adj_indices, adj_values, W, b):
    out = _linear(x, W.T, b.reshape(1, D))
    res = _spmm(out, adj_indices[0], adj_indices[1], adj_values)
    return res[:N]


# speedup vs baseline: 6.2065x; 1.1355x over previous
import functools

import jax
import jax.numpy as jnp
from jax import lax
from jax.experimental import pallas as pl
from jax.experimental.pallas import tpu as pltpu
from jax.experimental.pallas import tpu_sc as plsc

N = 10000
E = 320000
D = 128

NC = 2
NS = 16
HALF = 5120
ACC_H = HALF + 8
RPT = HALF // NS
EPT = E // NS
STAGE = 2000
NSTAGE = EPT // STAGE
GPS = STAGE // 16
K = 80
CROWS = 256
CAP = CROWS * K


def _linear_body(x_ref, wt_ref, b_ref, o_ref):
    o_ref[...] = (
        jnp.dot(x_ref[...], wt_ref[...], preferred_element_type=jnp.float32)
        + b_ref[...]
    )


_BM = 1000

_linear = pl.pallas_call(
    _linear_body,
    grid=(N // _BM,),
    in_specs=[
        pl.BlockSpec((_BM, D), lambda i: (i, 0)),
        pl.BlockSpec((D, D), lambda i: (0, 0)),
        pl.BlockSpec((1, D), lambda i: (0, 0)),
    ],
    out_specs=pl.BlockSpec((_BM, D), lambda i: (i, 0)),
    out_shape=jax.ShapeDtypeStruct((N, D), jnp.float32),
)


def _spmm_body(out_hbm, row_hbm, col_hbm, val_hbm, res_hbm,
               crow, ccol, cval, rrow, rcol, rval, rows0, rows1, acc,
               semg0, semg1, sems0, sems1):
    cid = lax.axis_index("c")
    sid = lax.axis_index("s")
    lo = cid * HALF

    zero16 = jnp.zeros((16,), jnp.float32)

    def zfill(r, carry):
        for j in range(D // 16):
            rows0[r, pl.ds(j * 16, 16)] = zero16
        return carry

    lax.fori_loop(0, K, zfill, 0)
    for t in range(RPT // K):
        pltpu.sync_copy(rows0, acc.at[pl.ds(sid * RPT + t * K, K)])

    @pl.when(sid == NS - 1)
    def _():
        pltpu.sync_copy(rows0.at[pl.ds(0, 8)], acc.at[pl.ds(HALF, 8)])

    dummy_row = jnp.full((16,), HALF, jnp.int32)
    zero_i = jnp.zeros((16,), jnp.int32)

    def pfill(p, carry):
        for j in range(K // 16):
            s2 = pl.ds(p * K + j * 16, 16)
            crow[s2] = dummy_row
            ccol[s2] = zero_i
            cval[s2] = zero16
        return carry

    lax.fori_loop(0, CROWS, pfill, 0)

    def stage_body(t, cnt):
        base = sid * EPT + t * STAGE
        pltpu.sync_copy(row_hbm.at[pl.ds(base, STAGE)], rrow)
        pltpu.sync_copy(col_hbm.at[pl.ds(base, STAGE)], rcol)
        pltpu.sync_copy(val_hbm.at[pl.ds(base, STAGE)], rval)

        def group_body(g, cnt):
            sl = pl.ds(g * 16, 16)
            rr = rrow[sl] - lo
            mask = (rr >= 0) & (rr < HALF)
            cs = plsc.cumsum(mask.astype(jnp.int32))
            pos = cnt + cs - 1
            plsc.store_scatter(crow, [pos], rr, mask=mask)
            plsc.store_scatter(ccol, [pos], rcol[sl], mask=mask)
            plsc.store_scatter(cval, [pos], rval[sl], mask=mask)
            return cnt + cs[15]

        return lax.fori_loop(0, GPS, group_body, cnt)

    cnt = lax.fori_loop(0, NSTAGE, stage_body, jnp.int32(0))

    plsc.subcore_barrier()

    nch2 = (cnt + (2 * K - 1)) // (2 * K)

    def issue_gather(c, buf, sem):
        return pltpu.async_copy(out_hbm.at[ccol.at[pl.ds(c * K, K)]], buf, sem)

    def scale(buf, c):
        for g in range(K // 16):
            val16 = cval[pl.ds(c * K + g * 16, 16)]
            for l in range(16):
                v = jnp.full((16,), val16[l], jnp.float32)
                e = g * 16 + l
                for j in range(D // 16):
                    sl = pl.ds(j * 16, 16)
                    buf[e, sl] = buf[e, sl] * v

    @pl.when(nch2 > 0)
    def _():
        issue_gather(0, rows0, semg0)
        issue_gather(1, rows1, semg1)
        return None

    def chunk_body(i, carry):
        a = 2 * i
        bc = 2 * i + 1
        pltpu.make_async_copy(
            out_hbm.at[ccol.at[pl.ds(a * K, K)]], rows0, semg0).wait()
        scale(rows0, a)
        h0 = []
        for g in range(K // 16):
            ridx = crow[pl.ds(a * K + g * 16, 16)]
            h0.append(pltpu.async_copy(rows0.at[pl.ds(g * 16, 16)],
                                       acc.at[ridx], sems0, add=True))

        pltpu.make_async_copy(
            out_hbm.at[ccol.at[pl.ds(bc * K, K)]], rows1, semg1).wait()
        scale(rows1, bc)
        h1 = []
        for g in range(K // 16):
            ridx = crow[pl.ds(bc * K + g * 16, 16)]
            h1.append(pltpu.async_copy(rows1.at[pl.ds(g * 16, 16)],
                                       acc.at[ridx], sems1, add=True))

        for h in h0:
            h.wait()

        @pl.when(i + 1 < nch2)
        def _():
            issue_gather(a + 2, rows0, semg0)

        for h in h1:
            h.wait()

        @pl.when(i + 1 < nch2)
        def _():
            issue_gather(bc + 2, rows1, semg1)

        return carry

    lax.fori_loop(0, nch2, chunk_body, 0)

    plsc.subcore_barrier()

    pltpu.sync_copy(acc.at[pl.ds(sid * RPT, RPT)],
                    res_hbm.at[pl.ds(lo + sid * RPT, RPT)])


_spmm = functools.partial(
    pl.kernel,
    out_type=jax.ShapeDtypeStruct((NC * HALF, D), jnp.float32),
    mesh=plsc.VectorSubcoreMesh(core_axis_name="c", subcore_axis_name="s"),
    compiler_params=pltpu.CompilerParams(needs_layout_passes=False),
    scratch_types=[
        pltpu.VMEM((CAP,), jnp.int32),
        pltpu.VMEM((CAP,), jnp.int32),
        pltpu.VMEM((CAP,), jnp.float32),
        pltpu.VMEM((STAGE,), jnp.int32),
        pltpu.VMEM((STAGE,), jnp.int32),
        pltpu.VMEM((STAGE,), jnp.float32),
        pltpu.VMEM((K, D), jnp.float32),
        pltpu.VMEM((K, D), jnp.float32),
        pltpu.VMEM_SHARED((ACC_H, D), jnp.float32),
        pltpu.SemaphoreType.DMA,
        pltpu.SemaphoreType.DMA,
        pltpu.SemaphoreType.DMA,
        pltpu.SemaphoreType.DMA,
    ],
)(_spmm_body)


def kernel(x, adj_indices, adj_values, W, b):
    out = _linear(x, W.T, b.reshape(1, D))
    res = _spmm(out, adj_indices[0], adj_indices[1], adj_values)
    return res[:N]
